# baseline (speedup 1.0000x reference)
import jax
import jax.numpy as jnp
from jax import lax
from jax.experimental import pallas as pl
from jax.experimental.pallas import tpu as pltpu

D_MODEL = 4096
N_EXPERTS = 64
TOP_K = 8
N_TOKENS = 8192
BT = 512
CHUNK = 2048
N_CHUNKS = N_TOKENS // CHUNK


def _topk_chunk(s):
    iota_f = lax.broadcasted_iota(jnp.int32, (CHUNK, N_EXPERTS), 1).astype(jnp.float32)
    e_iota = lax.broadcasted_iota(jnp.int32, (N_EXPERTS, 1), 0)
    pcol = lax.bitcast_convert_type(((127 - 2 * e_iota) << 23), jnp.float32)
    cur = s
    cols = []
    for _ in range(TOP_K):
        mx = jnp.max(cur, axis=1, keepdims=True)
        hf = jnp.where(cur == mx, 1.0, 0.0)
        pw = jnp.dot(hf, pcol, preferred_element_type=jnp.float32)
        bits = lax.bitcast_convert_type(pw, jnp.int32)
        idx = (127 - ((bits >> 23) & 0xFF)) >> 1
        cols.append(idx)
        idxf = idx.astype(jnp.float32)
        cur = jnp.where(iota_f == idxf, -jnp.inf, cur)
    return jnp.concatenate(cols, axis=1)


def _gate_body(x_ref, wt_ref, b_ref, out_ref, g_acc, m_acc, z_acc):
    i = pl.program_id(0)

    @pl.when(i == 0)
    def _():
        m_acc[...] = jnp.full((1, N_EXPERTS), -jnp.inf, jnp.float32)
        z_acc[...] = jnp.zeros((1, N_EXPERTS), jnp.float32)

    g = jnp.dot(x_ref[...], wt_ref[...], preferred_element_type=jnp.float32)
    g = g + b_ref[...]
    g_acc[pl.ds(i * BT, BT), :] = g

    m_old = m_acc[...]
    m_new = jnp.maximum(m_old, jnp.max(g, axis=0, keepdims=True))
    z_acc[...] = (z_acc[...] * jnp.exp(m_old - m_new)
                  + jnp.sum(jnp.exp(g - m_new), axis=0, keepdims=True))
    m_acc[...] = m_new

    @pl.when(i == pl.num_programs(0) - 1)
    def _():
        m = m_acc[...]
        z = z_acc[...]

        def tk_body(c, carry):
            blk = g_acc[pl.ds(c * CHUNK, CHUNK), :]
            s = jnp.exp(blk - m) / z
            out_ref[pl.ds(c * CHUNK, CHUNK), :] = _topk_chunk(s)
            return carry

        lax.fori_loop(0, N_CHUNKS, tk_body, 0)


def kernel(x, W, b):
    wt = W.T
    b2 = b.reshape(1, N_EXPERTS)
    grid = N_TOKENS // BT
    return pl.pallas_call(
        _gate_body,
        grid=(grid,),
        in_specs=[
            pl.BlockSpec((BT, D_MODEL), lambda i: (i, 0)),
            pl.BlockSpec((D_MODEL, N_EXPERTS), lambda i: (0, 0)),
            pl.BlockSpec((1, N_EXPERTS), lambda i: (0, 0)),
        ],
        out_specs=pl.BlockSpec((N_TOKENS, TOP_K), lambda i: (0, 0)),
        out_shape=jax.ShapeDtypeStruct((N_TOKENS, TOP_K), jnp.int32),
        scratch_shapes=[
            pltpu.VMEM((N_TOKENS, N_EXPERTS), jnp.float32),
            pltpu.VMEM((1, N_EXPERTS), jnp.float32),
            pltpu.VMEM((1, N_EXPERTS), jnp.float32),
        ],
    )(x, wt, b2)


# speedup vs baseline: 1.1206x; 1.1206x over previous
import jax
import jax.numpy as jnp
from jax import lax
from jax.experimental import pallas as pl
from jax.experimental.pallas import tpu as pltpu

D_MODEL = 4096
N_EXPERTS = 64
TOP_K = 8
N_TOKENS = 8192
BT = 512
CHUNK = 4096
N_CHUNKS = N_TOKENS // CHUNK


def _topk_chunk(s):
    iota_f = lax.broadcasted_iota(jnp.int32, (CHUNK, N_EXPERTS), 1).astype(jnp.float32)
    cur = s
    cols = []
    for _ in range(TOP_K):
        mx = jnp.max(cur, axis=1, keepdims=True)
        hit = cur == mx
        idxv = jnp.where(hit, iota_f, float(N_EXPERTS))
        idx = jnp.min(idxv, axis=1, keepdims=True)
        cols.append(idx)
        cur = jnp.where(idxv == idx, -jnp.inf, cur)
    return jnp.concatenate(cols, axis=1).astype(jnp.int32)


def _gate_body(x_ref, wt_ref, b_ref, out_ref, g_acc, m_acc, z_acc):
    i = pl.program_id(0)

    @pl.when(i == 0)
    def _():
        m_acc[...] = jnp.full((1, N_EXPERTS), -jnp.inf, jnp.float32)
        z_acc[...] = jnp.zeros((1, N_EXPERTS), jnp.float32)

    g = jnp.dot(x_ref[...], wt_ref[...], preferred_element_type=jnp.float32)
    g = g + b_ref[...]
    g_acc[pl.ds(i * BT, BT), :] = g

    m_old = m_acc[...]
    m_new = jnp.maximum(m_old, jnp.max(g, axis=0, keepdims=True))
    z_acc[...] = (z_acc[...] * jnp.exp(m_old - m_new)
                  + jnp.sum(jnp.exp(g - m_new), axis=0, keepdims=True))
    m_acc[...] = m_new

    @pl.when(i == pl.num_programs(0) - 1)
    def _():
        m = m_acc[...]
        z = z_acc[...]

        def tk_body(c, carry):
            blk = g_acc[pl.ds(c * CHUNK, CHUNK), :]
            s = jnp.exp(blk - m) / z
            out_ref[pl.ds(c * CHUNK, CHUNK), :] = _topk_chunk(s)
            return carry

        lax.fori_loop(0, N_CHUNKS, tk_body, 0)


def kernel(x, W, b):
    wt = W.T
    b2 = b.reshape(1, N_EXPERTS)
    grid = N_TOKENS // BT
    return pl.pallas_call(
        _gate_body,
        grid=(grid,),
        in_specs=[
            pl.BlockSpec((BT, D_MODEL), lambda i: (i, 0)),
            pl.BlockSpec((D_MODEL, N_EXPERTS), lambda i: (0, 0)),
            pl.BlockSpec((1, N_EXPERTS), lambda i: (0, 0)),
        ],
        out_specs=pl.BlockSpec((N_TOKENS, TOP_K), lambda i: (0, 0)),
        out_shape=jax.ShapeDtypeStruct((N_TOKENS, TOP_K), jnp.int32),
        scratch_shapes=[
            pltpu.VMEM((N_TOKENS, N_EXPERTS), jnp.float32),
            pltpu.VMEM((1, N_EXPERTS), jnp.float32),
            pltpu.VMEM((1, N_EXPERTS), jnp.float32),
        ],
    )(x, wt, b2)
